# baseline (speedup 1.0000x reference)
import functools

import jax
import jax.numpy as jnp
from jax import lax
from jax.experimental import pallas as pl
from jax.experimental.pallas import tpu as pltpu
from jax.experimental.pallas import tpu_sc as plsc

N, E, D, G = 10000, 320000, 128, 512
NP = 10240
NC, NS, L = 2, 16, 16
NW = NC * NS
ECH = 128
EP = 327680
NCHUNKS = EP // ECH
CPT = NCHUNKS // NW
RCH = 16
NRCH = NP // RCH
NXCH = N // RCH
BLK = 1024
NBLK = NP // BLK


def _sc_edge_agg(x, src2, dst2, ew2):
    mesh = plsc.VectorSubcoreMesh(core_axis_name="c", subcore_axis_name="s",
                                  num_cores=NC, num_subcores=NS)

    @functools.partial(
        pl.kernel,
        out_type=jax.ShapeDtypeStruct((NC, NP, D), jnp.float32),
        mesh=mesh,
        scratch_types=[
            pltpu.VMEM((4 * ECH,), jnp.int32),
            pltpu.VMEM((4, 1, ECH), jnp.int32),
            pltpu.VMEM((4 * ECH,), jnp.float32),
            pltpu.VMEM((ECH, D), jnp.float32),
            pltpu.VMEM((ECH, D), jnp.float32),
            pltpu.VMEM((RCHZ, D), jnp.float32),
            pltpu.VMEM_SHARED((NP, D), jnp.float32),
            pltpu.SemaphoreType.DMA,
            pltpu.SemaphoreType.DMA,
            pltpu.SemaphoreType.DMA,
            pltpu.SemaphoreType.DMA,
        ],
    )
    def k(x_hbm, src_hbm, dst_hbm, ew_hbm, p_hbm,
          src_v, dst_v, ew_v, rows0, rows1, zrow_v, agg_sh, gs0, gs1,
          is0, is1):
        cid = lax.axis_index("c")
        sid = lax.axis_index("s")
        wid = sid * NC + cid
        zero = jnp.zeros((L,), jnp.float32)

        def zfill(r, c):
            for cb in range(D // L):
                zrow_v[r, pl.ds(cb * L, L)] = zero
            return c

        lax.fori_loop(0, RCHZ, zfill, 0)

        @pl.when(jnp.logical_and(cid == 0, sid < NS - 1))
        def _():
            r0 = sid * XRT
            pltpu.sync_copy(x_hbm.at[pl.ds(r0, XRT)],
                            agg_sh.at[pl.ds(r0, XRT)])

        @pl.when(jnp.logical_and(cid == 0, sid == NS - 1))
        def _():
            r0 = (NS - 1) * XRT
            pltpu.sync_copy(x_hbm.at[pl.ds(r0, N - r0)],
                            agg_sh.at[pl.ds(r0, N - r0)])

        @pl.when(cid == 0)
        def _():
            @pl.when(sid < (NP - N) // 16)
            def _():
                pltpu.sync_copy(zrow_v.at[pl.ds(0, 16)],
                                agg_sh.at[pl.ds(N + sid * 16, 16)])

        @pl.when(cid == 1)
        def _():
            def zc(i, c):
                r0 = (i * NS + sid) * RCHZ
                pltpu.sync_copy(zrow_v, agg_sh.at[pl.ds(r0, RCHZ)])
                return c

            lax.fori_loop(0, NP // RCHZ // NS, zc, 0)

        plsc.subcore_barrier()

        base = wid * CPT * ECH
        baser = wid * CPT
        rows = (rows0, rows1)
        gsems = (gs0, gs1)
        isems = (is0, is1)

        def fire_idx(chunk, slot, sem):
            pltpu.async_copy(src_hbm.at[pl.ds(base + chunk * ECH, ECH)],
                             src_v.at[pl.ds(slot * ECH, ECH)], sem)
            pltpu.async_copy(dst_hbm.at[pl.ds(baser + chunk, 1)],
                             dst_v.at[pl.ds(slot, 1)], sem)
            pltpu.async_copy(ew_hbm.at[pl.ds(base + chunk * ECH, ECH)],
                             ew_v.at[pl.ds(slot * ECH, ECH)], sem)

        def wait_idx(slot, sem):
            pltpu.make_async_copy(src_hbm.at[pl.ds(base, ECH)],
                                  src_v.at[pl.ds(slot * ECH, ECH)], sem).wait()
            pltpu.make_async_copy(dst_hbm.at[pl.ds(baser, 1)],
                                  dst_v.at[pl.ds(slot, 1)], sem).wait()
            pltpu.make_async_copy(ew_hbm.at[pl.ds(base, ECH)],
                                  ew_v.at[pl.ds(slot * ECH, ECH)], sem).wait()

        fire_idx(0, 0, is0)
        fire_idx(1, 1, is1)
        wait_idx(0, is0)
        pltpu.async_copy(x_hbm.at[src_v.at[pl.ds(0, ECH)]], rows0, gs0)
        wait_idx(1, is1)
        pltpu.async_copy(x_hbm.at[src_v.at[pl.ds(ECH, ECH)]], rows1, gs1)
        fire_idx(2, 2, is0)
        fire_idx(3, 3, is1)

        dnums = lax.GatherDimensionNumbers(
            offset_dims=(), collapsed_slice_dims=(0,), start_index_map=(0,))

        def chunk_pair(k2, c):
            for b in range(2):
                kk = k2 * 2 + b
                slot = jnp.bitwise_and(kk, 3)
                rv = rows[b]
                gsem = gsems[b]
                isem = isems[b]

                pltpu.make_async_copy(
                    x_hbm.at[src_v.at[pl.ds(slot * ECH, ECH)]], rv,
                    gsem).wait()

                def grp(g, c2):
                    r0 = g * L
                    ewv = ew_v[pl.ds(slot * ECH + r0, L)]
                    for j in range(L):
                        w = lax.gather(
                            ewv, jnp.full((L, 1), j, jnp.int32), dnums, (1,),
                            mode=lax.GatherScatterMode.PROMISE_IN_BOUNDS)
                        for cb in range(D // L):
                            sl = pl.ds(cb * L, L)
                            rv[r0 + j, sl] = rv[r0 + j, sl] * w
                    return c2

                lax.fori_loop(0, ECH // L, grp, 0)
                pltpu.sync_copy(rv, agg_sh.at[dst_v.at[slot, 0]], add=True)

                slot2 = jnp.bitwise_and(kk + 2, 3)

                @pl.when(kk + 2 < CPT)
                def _():
                    wait_idx(slot2, isem)
                    pltpu.async_copy(
                        x_hbm.at[src_v.at[pl.ds(slot2 * ECH, ECH)]], rv, gsem)

                @pl.when(kk + 4 < CPT)
                def _():
                    fire_idx(kk + 4, slot, isem)

            return c

        lax.fori_loop(0, CPT // 2, chunk_pair, 0)
        plsc.subcore_barrier()

        r0 = sid * (NP // NS)
        pltpu.sync_copy(agg_sh.at[pl.ds(r0, NP // NS)],
                        p_hbm.at[cid, pl.ds(r0, NP // NS)])

    return k(x, src2, dst2, ew2)


def _tc_post(p, batch3, W1, W2, Wp, bp2, gamma2, beta2, alpha2):
    def body(p_ref, b_ref, W1_ref, W2_ref, Wp_ref, bp_ref, gam_ref, bet_ref,
             al_ref, g_ref, gt_ref, hp_ref, sagg):
        i = pl.program_id(0)
        agg = p_ref[0] + p_ref[1]
        h_on = jnp.dot(agg, W2_ref[...], preferred_element_type=jnp.float32)
        z = jnp.dot(h_on, Wp_ref[...], preferred_element_type=jnp.float32)
        z = z + bp_ref[...]
        mu = jnp.mean(z, axis=-1, keepdims=True)
        var = jnp.mean((z - mu) ** 2, axis=-1, keepdims=True)
        z = (z - mu) / jnp.sqrt(var + 1e-5) * gam_ref[...] + bet_ref[...]
        alpha = al_ref[0, 0]
        hp_ref[...] = jnp.where(z >= 0, z, alpha * z)

        bvec = b_ref[0, 0, :]
        seg = lax.broadcasted_iota(jnp.int32, (G, BLK), 0)
        mask = (bvec[None, :] == seg).astype(jnp.float32)
        part = jnp.dot(mask, agg, preferred_element_type=jnp.float32)

        @pl.when(i == 0)
        def _():
            sagg[...] = part

        @pl.when(i > 0)
        def _():
            sagg[...] = sagg[...] + part

        @pl.when(i == NBLK - 1)
        def _():
            s = sagg[...]
            g_ref[...] = jnp.dot(s, W1_ref[...], preferred_element_type=jnp.float32)
            gt_ref[...] = jnp.dot(s, W2_ref[...], preferred_element_type=jnp.float32)

    return pl.pallas_call(
        body,
        grid=(NBLK,),
        in_specs=[
            pl.BlockSpec((NC, BLK, D), lambda i: (0, i, 0)),
            pl.BlockSpec((1, 1, BLK), lambda i: (i, 0, 0)),
            pl.BlockSpec((D, D), lambda i: (0, 0)),
            pl.BlockSpec((D, D), lambda i: (0, 0)),
            pl.BlockSpec((D, D), lambda i: (0, 0)),
            pl.BlockSpec((1, D), lambda i: (0, 0)),
            pl.BlockSpec((1, D), lambda i: (0, 0)),
            pl.BlockSpec((1, D), lambda i: (0, 0)),
            pl.BlockSpec((1, 1), lambda i: (0, 0)),
        ],
        out_specs=[
            pl.BlockSpec((G, D), lambda i: (0, 0)),
            pl.BlockSpec((G, D), lambda i: (0, 0)),
            pl.BlockSpec((BLK, D), lambda i: (i, 0)),
        ],
        out_shape=[
            jax.ShapeDtypeStruct((G, D), jnp.float32),
            jax.ShapeDtypeStruct((G, D), jnp.float32),
            jax.ShapeDtypeStruct((NP, D), jnp.float32),
        ],
        scratch_shapes=[pltpu.VMEM((G, D), jnp.float32)],
    )(p, batch3, W1, W2, Wp, bp2, gamma2, beta2, alpha2)


def kernel(x, edge_index, edge_weight, batch, W1, W2, Wp, bp, gamma, beta, alpha):
    src1 = jnp.pad(edge_index[0], (0, EP - E))
    dst1 = jnp.pad(edge_index[1], (0, EP - E))
    ew1 = jnp.pad(edge_weight, (0, EP - E))
    p = _sc_edge_agg(x, src1, dst1, ew1)
    batch_pad = jnp.concatenate([batch, jnp.full((NP - N,), G, jnp.int32)])
    batch3 = batch_pad.reshape(NBLK, 1, BLK)
    g, gt, hp = _tc_post(
        p, batch3, W1, W2, Wp,
        bp.reshape(1, D), gamma.reshape(1, D), beta.reshape(1, D),
        jnp.asarray(alpha, jnp.float32).reshape(1, 1),
    )
    h_pred = hp[:N]
    return (g, g, h_pred, h_pred, gt, gt)
